# baseline (speedup 1.0000x reference)
import functools

import jax
import jax.numpy as jnp
from jax import lax
from jax.experimental import pallas as pl
from jax.experimental.pallas import tpu as pltpu
from jax.experimental.pallas import tpu_sc as plsc

N = 10000
E = 320000
D = 128
NC, NS = 2, 16
NW = NC * NS
EPW = E // NW
CHUNK = 100
NCHUNK = EPW // CHUNK
RPS = 624
RPS_LAST = N - (NS - 1) * RPS


def _sc_aggregate(x, edges_r, zeros):
    mesh = plsc.VectorSubcoreMesh(core_axis_name="c", subcore_axis_name="s")

    @functools.partial(
        pl.kernel,
        out_type=jax.ShapeDtypeStruct((NC, N, D), jnp.float32),
        mesh=mesh,
        compiler_params=pltpu.CompilerParams(use_tc_tiling_on_sc=False),
        scratch_types=[
            pltpu.VMEM((2, NCHUNK, CHUNK), jnp.int32),
            pltpu.VMEM((CHUNK, D), jnp.float32),
            pltpu.VMEM((CHUNK, D), jnp.float32),
            pltpu.VMEM_SHARED((N, D), jnp.float32),
            pltpu.SemaphoreType.DMA,
            pltpu.SemaphoreType.DMA,
            pltpu.SemaphoreType.DMA,
            pltpu.SemaphoreType.DMA,
        ],
    )
    def k(x_hbm, e_hbm, z_hbm, out_hbm, idx_v, rows0_v, rows1_v, agg_sh,
          sem0, sem1, ssem0, ssem1):
        c = lax.axis_index("c")
        s = lax.axis_index("s")
        wid = c * NS + s
        pltpu.sync_copy(e_hbm.at[0, wid], idx_v.at[0])
        pltpu.sync_copy(e_hbm.at[1, wid], idx_v.at[1])
        r0 = s * RPS

        @pl.when(s < NS - 1)
        def _():
            pltpu.sync_copy(z_hbm.at[pl.ds(0, RPS)], agg_sh.at[pl.ds(r0, RPS)])

        @pl.when(s == NS - 1)
        def _():
            pltpu.sync_copy(
                z_hbm.at[pl.ds(0, RPS_LAST)],
                agg_sh.at[pl.ds((NS - 1) * RPS, RPS_LAST)],
            )

        plsc.subcore_barrier()

        last = NCHUNK - 1

        def gather(j, buf, sem):
            pltpu.async_copy(x_hbm.at[idx_v.at[0, j]], buf, sem)

        def wait(buf, sem):
            pltpu.make_async_copy(x_hbm.at[pl.ds(0, CHUNK)], buf, sem).wait()

        def scatter(j, buf, sem):
            pltpu.async_copy(buf, agg_sh.at[idx_v.at[1, j]], sem, add=True)

        def swait(buf, sem):
            pltpu.make_async_copy(buf, agg_sh.at[pl.ds(0, CHUNK)], sem).wait()

        gather(0, rows0_v, sem0)
        gather(1, rows1_v, sem1)

        def body(i, carry):
            j0 = 2 * i
            wait(rows0_v, sem0)
            scatter(j0, rows0_v, ssem0)
            wait(rows1_v, sem1)
            scatter(j0 + 1, rows1_v, ssem1)
            swait(rows0_v, ssem0)
            gather(jnp.minimum(j0 + 2, last), rows0_v, sem0)
            swait(rows1_v, ssem1)
            gather(jnp.minimum(j0 + 3, last), rows1_v, sem1)
            return carry

        lax.fori_loop(0, NCHUNK // 2, body, 0)
        wait(rows0_v, sem0)
        wait(rows1_v, sem1)
        plsc.subcore_barrier()

        @pl.when(s < NS - 1)
        def _():
            pltpu.sync_copy(
                agg_sh.at[pl.ds(r0, RPS)], out_hbm.at[c, pl.ds(r0, RPS)]
            )

        @pl.when(s == NS - 1)
        def _():
            pltpu.sync_copy(
                agg_sh.at[pl.ds((NS - 1) * RPS, RPS_LAST)],
                out_hbm.at[c, pl.ds((NS - 1) * RPS, RPS_LAST)],
            )

    return k(x, edges_r, zeros)


def _tc_finish(x, parts, w2d, W1, b1, W2, b2):
    def body(x_ref, p_ref, w_ref, w1_ref, b1_ref, w2_ref, b2_ref, out_ref):
        h = x_ref[...] + p_ref[0] + p_ref[1]
        t = jnp.dot(h, w1_ref[...].T, preferred_element_type=jnp.float32)
        t = jnp.maximum(t + b1_ref[...], 0.0)
        wv = w_ref[...]
        v = jnp.sum(t * wv, axis=0, keepdims=True)
        sw = jnp.sum(wv)
        out = jnp.dot(v, w2_ref[...].T, preferred_element_type=jnp.float32)
        out_ref[...] = out + sw * b2_ref[...]

    return pl.pallas_call(
        body,
        out_shape=jax.ShapeDtypeStruct((1, D), jnp.float32),
    )(x, parts, w2d, W1, b1, W2, b2)


def kernel(x, edge_index, weights, W1, b1, W2, b2):
    edges_r = edge_index.reshape(2, NW, NCHUNK, CHUNK)
    zeros = jnp.zeros((RPS_LAST, D), jnp.float32)
    parts = _sc_aggregate(x, edges_r, zeros)
    out = _tc_finish(x, parts, weights.reshape(N, 1), W1, b1, W2, b2)
    return out.reshape(1, 1, D)


# speedup vs baseline: 1.2272x; 1.2272x over previous
import functools

import jax
import jax.numpy as jnp
from jax import lax
from jax.experimental import pallas as pl
from jax.experimental.pallas import tpu as pltpu
from jax.experimental.pallas import tpu_sc as plsc

N = 10000
E = 320000
D = 128
NC, NS = 2, 16
NW = NC * NS
EPW = E // NW
CHUNK = 100
NCHUNK = EPW // CHUNK
RPS = 624
RPS_LAST = N - (NS - 1) * RPS


def _sc_aggregate(x, edges_r, zeros):
    mesh = plsc.VectorSubcoreMesh(core_axis_name="c", subcore_axis_name="s")

    @functools.partial(
        pl.kernel,
        out_type=jax.ShapeDtypeStruct((NC, N, D), jnp.float32),
        mesh=mesh,
        compiler_params=pltpu.CompilerParams(use_tc_tiling_on_sc=False),
        scratch_types=[
            pltpu.VMEM((2, NCHUNK, CHUNK), jnp.int32),
            pltpu.VMEM((CHUNK, D), jnp.float32),
            pltpu.VMEM((CHUNK, D), jnp.float32),
            pltpu.VMEM_SHARED((N, D), jnp.float32),
            pltpu.SemaphoreType.DMA,
            pltpu.SemaphoreType.DMA,
        ],
    )
    def k(x_hbm, e_hbm, z_hbm, out_hbm, idx_v, rows0_v, rows1_v, agg_sh,
          sem0, sem1):
        c = lax.axis_index("c")
        s = lax.axis_index("s")
        wid = c * NS + s
        pltpu.sync_copy(e_hbm.at[0, wid], idx_v.at[0])
        pltpu.sync_copy(e_hbm.at[1, wid], idx_v.at[1])
        r0 = s * RPS

        @pl.when(s < NS - 1)
        def _():
            pltpu.sync_copy(z_hbm.at[pl.ds(0, RPS)], agg_sh.at[pl.ds(r0, RPS)])

        @pl.when(s == NS - 1)
        def _():
            pltpu.sync_copy(
                z_hbm.at[pl.ds(0, RPS_LAST)],
                agg_sh.at[pl.ds((NS - 1) * RPS, RPS_LAST)],
            )

        plsc.subcore_barrier()

        last = NCHUNK - 1

        def gather(j, buf, sem):
            pltpu.async_copy(x_hbm.at[idx_v.at[0, j]], buf, sem)

        def wait(buf, sem):
            pltpu.make_async_copy(x_hbm.at[pl.ds(0, CHUNK)], buf, sem).wait()

        gather(0, rows0_v, sem0)
        gather(1, rows1_v, sem1)

        def body(i, carry):
            j0 = 2 * i
            wait(rows0_v, sem0)
            pltpu.sync_copy(rows0_v, agg_sh.at[idx_v.at[1, j0]], add=True)
            gather(jnp.minimum(j0 + 2, last), rows0_v, sem0)
            wait(rows1_v, sem1)
            pltpu.sync_copy(rows1_v, agg_sh.at[idx_v.at[1, j0 + 1]], add=True)
            gather(jnp.minimum(j0 + 3, last), rows1_v, sem1)
            return carry

        lax.fori_loop(0, NCHUNK // 2, body, 0)
        wait(rows0_v, sem0)
        wait(rows1_v, sem1)
        plsc.subcore_barrier()

        @pl.when(s < NS - 1)
        def _():
            pltpu.sync_copy(
                agg_sh.at[pl.ds(r0, RPS)], out_hbm.at[c, pl.ds(r0, RPS)]
            )

        @pl.when(s == NS - 1)
        def _():
            pltpu.sync_copy(
                agg_sh.at[pl.ds((NS - 1) * RPS, RPS_LAST)],
                out_hbm.at[c, pl.ds((NS - 1) * RPS, RPS_LAST)],
            )

    return k(x, edges_r, zeros)


def _tc_finish(x, parts, w2d, W1, b1, W2, b2):
    def body(x_ref, p_ref, w_ref, w1_ref, b1_ref, w2_ref, b2_ref, out_ref):
        h = x_ref[...] + p_ref[0] + p_ref[1]
        t = jnp.dot(h, w1_ref[...].T, preferred_element_type=jnp.float32)
        t = jnp.maximum(t + b1_ref[...], 0.0)
        wv = w_ref[...]
        v = jnp.sum(t * wv, axis=0, keepdims=True)
        sw = jnp.sum(wv)
        out = jnp.dot(v, w2_ref[...].T, preferred_element_type=jnp.float32)
        out_ref[...] = out + sw * b2_ref[...]

    return pl.pallas_call(
        body,
        out_shape=jax.ShapeDtypeStruct((1, D), jnp.float32),
    )(x, parts, w2d, W1, b1, W2, b2)


def kernel(x, edge_index, weights, W1, b1, W2, b2):
    edges_r = edge_index.reshape(2, NW, NCHUNK, CHUNK)
    zeros = jnp.zeros((RPS_LAST, D), jnp.float32)
    parts = _sc_aggregate(x, edges_r, zeros)
    out = _tc_finish(x, parts, weights.reshape(N, 1), W1, b1, W2, b2)
    return out.reshape(1, 1, D)


# speedup vs baseline: 1.2432x; 1.0131x over previous
import functools

import jax
import jax.numpy as jnp
from jax import lax
from jax.experimental import pallas as pl
from jax.experimental.pallas import tpu as pltpu
from jax.experimental.pallas import tpu_sc as plsc

N = 10000
E = 320000
D = 128
NC, NS = 2, 16
NW = NC * NS
EPW = E // NW
CHUNK = 100
NCHUNK = EPW // CHUNK
RPS = 624
RPS_LAST = N - (NS - 1) * RPS


def _sc_aggregate(x, edges_r, zeros):
    mesh = plsc.VectorSubcoreMesh(core_axis_name="c", subcore_axis_name="s")

    @functools.partial(
        pl.kernel,
        out_type=jax.ShapeDtypeStruct((NC, N, D), jnp.float32),
        mesh=mesh,
        compiler_params=pltpu.CompilerParams(use_tc_tiling_on_sc=False),
        scratch_types=[
            pltpu.VMEM((2, NCHUNK, CHUNK), jnp.int32),
            pltpu.VMEM((CHUNK, D), jnp.float32),
            pltpu.VMEM((CHUNK, D), jnp.float32),
            pltpu.VMEM_SHARED((N, D), jnp.float32),
            pltpu.SemaphoreType.DMA,
            pltpu.SemaphoreType.DMA,
        ],
    )
    def k(x_hbm, e_hbm, z_hbm, out_hbm, idx_v, rows0_v, rows1_v, agg_sh,
          sem0, sem1):
        c = lax.axis_index("c")
        s = lax.axis_index("s")
        wid = c * NS + s
        pltpu.sync_copy(e_hbm.at[0, wid], idx_v.at[0])
        pltpu.sync_copy(e_hbm.at[1, wid], idx_v.at[1])
        r0 = s * RPS

        @pl.when(s < NS - 1)
        def _():
            pltpu.sync_copy(z_hbm.at[pl.ds(0, RPS)], agg_sh.at[pl.ds(r0, RPS)])

        @pl.when(s == NS - 1)
        def _():
            pltpu.sync_copy(
                z_hbm.at[pl.ds(0, RPS_LAST)],
                agg_sh.at[pl.ds((NS - 1) * RPS, RPS_LAST)],
            )

        plsc.subcore_barrier()

        last = NCHUNK - 1

        def gather(j, buf, sem):
            pltpu.async_copy(x_hbm.at[idx_v.at[0, j]], buf, sem)

        def wait(buf, sem):
            pltpu.make_async_copy(x_hbm.at[pl.ds(0, CHUNK)], buf, sem).wait()

        gather(0, rows0_v, sem0)
        gather(1, rows1_v, sem1)

        def body(i, carry):
            j0 = 2 * i
            wait(rows0_v, sem0)
            pltpu.sync_copy(rows0_v, agg_sh.at[idx_v.at[1, j0]], add=True)
            gather(jnp.minimum(j0 + 2, last), rows0_v, sem0)
            wait(rows1_v, sem1)
            pltpu.sync_copy(rows1_v, agg_sh.at[idx_v.at[1, j0 + 1]], add=True)
            gather(jnp.minimum(j0 + 3, last), rows1_v, sem1)
            return carry

        lax.fori_loop(0, NCHUNK // 2, body, 0)
        wait(rows0_v, sem0)
        wait(rows1_v, sem1)
        plsc.subcore_barrier()

        @pl.when(s < NS - 1)
        def _():
            pltpu.sync_copy(
                agg_sh.at[pl.ds(r0, RPS)], out_hbm.at[c, pl.ds(r0, RPS)]
            )

        @pl.when(s == NS - 1)
        def _():
            pltpu.sync_copy(
                agg_sh.at[pl.ds((NS - 1) * RPS, RPS_LAST)],
                out_hbm.at[c, pl.ds((NS - 1) * RPS, RPS_LAST)],
            )

    return k(x, edges_r, zeros)


TC_BLK = 2000
TC_G = N // TC_BLK


def _tc_finish(x, parts, w2d, W1, b1, W2, b2):
    def body(x_ref, p_ref, w_ref, w1_ref, b1_ref, w2_ref, b2_ref, out_ref,
             acc_ref):
        g = pl.program_id(0)
        h = x_ref[...] + p_ref[0] + p_ref[1]
        t = jnp.dot(h, w1_ref[...].T, preferred_element_type=jnp.float32)
        t = jnp.maximum(t + b1_ref[...], 0.0)
        wv = w_ref[...]
        v = jnp.sum(t * wv, axis=0, keepdims=True)
        sw = jnp.sum(wv)

        @pl.when(g == 0)
        def _():
            acc_ref[...] = jnp.zeros_like(acc_ref)

        acc_ref[0:1, :] += v
        acc_ref[1:2, :] += sw

        @pl.when(g == TC_G - 1)
        def _():
            vv = acc_ref[0:1, :]
            out = jnp.dot(vv, w2_ref[...].T, preferred_element_type=jnp.float32)
            out_ref[...] = out + acc_ref[1, 0] * b2_ref[...]

    return pl.pallas_call(
        body,
        grid=(TC_G,),
        in_specs=[
            pl.BlockSpec((TC_BLK, D), lambda g: (g, 0)),
            pl.BlockSpec((NC, TC_BLK, D), lambda g: (0, g, 0)),
            pl.BlockSpec((TC_BLK, 1), lambda g: (g, 0)),
            pl.BlockSpec((D, D), lambda g: (0, 0)),
            pl.BlockSpec((D,), lambda g: (0,)),
            pl.BlockSpec((D, D), lambda g: (0, 0)),
            pl.BlockSpec((D,), lambda g: (0,)),
        ],
        out_specs=pl.BlockSpec((1, D), lambda g: (0, 0)),
        out_shape=jax.ShapeDtypeStruct((1, D), jnp.float32),
        scratch_shapes=[pltpu.VMEM((2, D), jnp.float32)],
    )(x, parts, w2d, W1, b1, W2, b2)


def kernel(x, edge_index, weights, W1, b1, W2, b2):
    edges_r = edge_index.reshape(2, NW, NCHUNK, CHUNK)
    zeros = jnp.zeros((RPS_LAST, D), jnp.float32)
    parts = _sc_aggregate(x, edges_r, zeros)
    out = _tc_finish(x, parts, weights.reshape(N, 1), W1, b1, W2, b2)
    return out.reshape(1, 1, D)


# speedup vs baseline: 1.2526x; 1.0076x over previous
import functools

import jax
import jax.numpy as jnp
from jax import lax
from jax.experimental import pallas as pl
from jax.experimental.pallas import tpu as pltpu
from jax.experimental.pallas import tpu_sc as plsc

N = 10000
E = 320000
D = 128
NC, NS = 2, 16
NW = NC * NS
EPW = E // NW
CHUNK = 100
NCHUNK = EPW // CHUNK
RPS = 624
RPS_LAST = N - (NS - 1) * RPS


def _sc_aggregate(x, edges_r, zeros):
    mesh = plsc.VectorSubcoreMesh(core_axis_name="c", subcore_axis_name="s")

    @functools.partial(
        pl.kernel,
        out_type=jax.ShapeDtypeStruct((NC, N, D), jnp.float32),
        mesh=mesh,
        compiler_params=pltpu.CompilerParams(use_tc_tiling_on_sc=False),
        scratch_types=[
            pltpu.VMEM((2, NCHUNK, CHUNK), jnp.int32),
            pltpu.VMEM((CHUNK, D), jnp.float32),
            pltpu.VMEM((CHUNK, D), jnp.float32),
            pltpu.VMEM_SHARED((N, D), jnp.float32),
            pltpu.SemaphoreType.DMA,
            pltpu.SemaphoreType.DMA,
        ],
    )
    def k(x_hbm, e_hbm, z_hbm, out_hbm, idx_v, rows0_v, rows1_v, agg_sh,
          sem0, sem1):
        c = lax.axis_index("c")
        s = lax.axis_index("s")
        wid = c * NS + s
        r0 = s * RPS
        pltpu.async_copy(e_hbm.at[0, wid], idx_v.at[0], sem0)
        pltpu.async_copy(e_hbm.at[1, wid], idx_v.at[1], sem1)

        @pl.when(s < NS - 1)
        def _():
            pltpu.sync_copy(z_hbm.at[pl.ds(0, RPS)], agg_sh.at[pl.ds(r0, RPS)])

        @pl.when(s == NS - 1)
        def _():
            pltpu.sync_copy(
                z_hbm.at[pl.ds(0, RPS_LAST)],
                agg_sh.at[pl.ds((NS - 1) * RPS, RPS_LAST)],
            )

        pltpu.make_async_copy(e_hbm.at[0, wid], idx_v.at[0], sem0).wait()
        pltpu.make_async_copy(e_hbm.at[1, wid], idx_v.at[1], sem1).wait()
        plsc.subcore_barrier()

        last = NCHUNK - 1

        def gather(j, buf, sem):
            pltpu.async_copy(x_hbm.at[idx_v.at[0, j]], buf, sem)

        def wait(buf, sem):
            pltpu.make_async_copy(x_hbm.at[pl.ds(0, CHUNK)], buf, sem).wait()

        gather(0, rows0_v, sem0)
        gather(1, rows1_v, sem1)

        def body(i, carry):
            j0 = 2 * i
            wait(rows0_v, sem0)
            pltpu.sync_copy(rows0_v, agg_sh.at[idx_v.at[1, j0]], add=True)
            gather(jnp.minimum(j0 + 2, last), rows0_v, sem0)
            wait(rows1_v, sem1)
            pltpu.sync_copy(rows1_v, agg_sh.at[idx_v.at[1, j0 + 1]], add=True)
            gather(jnp.minimum(j0 + 3, last), rows1_v, sem1)
            return carry

        lax.fori_loop(0, NCHUNK // 2, body, 0)
        wait(rows0_v, sem0)
        wait(rows1_v, sem1)
        plsc.subcore_barrier()

        @pl.when(s < NS - 1)
        def _():
            pltpu.sync_copy(
                agg_sh.at[pl.ds(r0, RPS)], out_hbm.at[c, pl.ds(r0, RPS)]
            )

        @pl.when(s == NS - 1)
        def _():
            pltpu.sync_copy(
                agg_sh.at[pl.ds((NS - 1) * RPS, RPS_LAST)],
                out_hbm.at[c, pl.ds((NS - 1) * RPS, RPS_LAST)],
            )

    return k(x, edges_r, zeros)


TC_BLK = 2000
TC_G = N // TC_BLK


def _tc_finish(x, parts, w2d, W1, b1, W2, b2):
    def body(x_ref, p_ref, w_ref, w1_ref, b1_ref, w2_ref, b2_ref, out_ref,
             acc_ref):
        g = pl.program_id(0)
        h = x_ref[...] + p_ref[0] + p_ref[1]
        t = jnp.dot(h, w1_ref[...].T, preferred_element_type=jnp.float32)
        t = jnp.maximum(t + b1_ref[...], 0.0)
        wv = w_ref[...]
        v = jnp.sum(t * wv, axis=0, keepdims=True)
        sw = jnp.sum(wv)

        @pl.when(g == 0)
        def _():
            acc_ref[...] = jnp.zeros_like(acc_ref)

        acc_ref[0:1, :] += v
        acc_ref[1:2, :] += sw

        @pl.when(g == TC_G - 1)
        def _():
            vv = acc_ref[0:1, :]
            out = jnp.dot(vv, w2_ref[...].T, preferred_element_type=jnp.float32)
            out_ref[...] = out + acc_ref[1, 0] * b2_ref[...]

    return pl.pallas_call(
        body,
        grid=(TC_G,),
        in_specs=[
            pl.BlockSpec((TC_BLK, D), lambda g: (g, 0)),
            pl.BlockSpec((NC, TC_BLK, D), lambda g: (0, g, 0)),
            pl.BlockSpec((TC_BLK, 1), lambda g: (g, 0)),
            pl.BlockSpec((D, D), lambda g: (0, 0)),
            pl.BlockSpec((D,), lambda g: (0,)),
            pl.BlockSpec((D, D), lambda g: (0, 0)),
            pl.BlockSpec((D,), lambda g: (0,)),
        ],
        out_specs=pl.BlockSpec((1, D), lambda g: (0, 0)),
        out_shape=jax.ShapeDtypeStruct((1, D), jnp.float32),
        scratch_shapes=[pltpu.VMEM((2, D), jnp.float32)],
    )(x, parts, w2d, W1, b1, W2, b2)


def kernel(x, edge_index, weights, W1, b1, W2, b2):
    edges_r = edge_index.reshape(2, NW, NCHUNK, CHUNK)
    zeros = jnp.zeros((RPS_LAST, D), jnp.float32)
    parts = _sc_aggregate(x, edges_r, zeros)
    out = _tc_finish(x, parts, weights.reshape(N, 1), W1, b1, W2, b2)
    return out.reshape(1, 1, D)
